# baseline (speedup 1.0000x reference)
import functools

import jax
import jax.numpy as jnp
from jax import lax
from jax.experimental import pallas as pl
from jax.experimental.pallas import tpu as pltpu
from jax.experimental.pallas import tpu_sc as plsc

N = 10000
E = 320000
DIN = 128
DH = 256
DOUT = 64

NC = 2
NS = 16
NACC = 10240
RPT = NACC // NS
EB = 80
EPT = E // NS
NB = EPT // EB
NBW = NB

_mesh = plsc.VectorSubcoreMesh(core_axis_name="c", subcore_axis_name="s")


@functools.partial(
    pl.kernel,
    out_type=jax.ShapeDtypeStruct((NACC,), jnp.float32),
    mesh=_mesh,
    compiler_params=pltpu.CompilerParams(use_tc_tiling_on_sc=False),
    scratch_types=[
        pltpu.VMEM_SHARED((NACC,), jnp.float32),
        pltpu.VMEM((RPT,), jnp.float32),
        pltpu.VMEM((NBW, EB), jnp.int32),
        pltpu.VMEM((NBW, EB), jnp.float32),
        pltpu.SemaphoreType.DMA,
    ],
)
def _deg_kernel(dst2w_hbm, ew2w_hbm, out_hbm, acc, buf, didx, ewb, sem):
    cid = lax.axis_index("c")
    sid = lax.axis_index("s")
    for j in range(RPT // 16):
        buf[pl.ds(j * 16, 16)] = jnp.zeros((16,), jnp.float32)
    pltpu.sync_copy(buf, acc.at[pl.ds(sid * RPT, RPT)])

    pltpu.sync_copy(dst2w_hbm.at[pl.ds(sid * NBW, NBW), :], didx)
    pltpu.sync_copy(ew2w_hbm.at[pl.ds(sid * NBW, NBW), :], ewb)
    plsc.subcore_barrier()

    K = 5

    @pl.loop(0, NBW, step=K)
    def _edge_group(i):
        for k in range(K):
            pltpu.async_copy(ewb.at[i + k], acc.at[didx.at[i + k]], sem,
                             add=True)
        for k in range(K):
            pltpu.make_async_copy(ewb.at[i + k], acc.at[didx.at[i + k]],
                                  sem).wait()

    plsc.subcore_barrier()

    @pl.when(cid == 0)
    def _wb():
        pltpu.sync_copy(acc.at[pl.ds(sid * RPT, RPT)], buf)
        pltpu.sync_copy(buf, out_hbm.at[pl.ds(sid * RPT, RPT)])


def _make_agg(F, plan, NG, NS_):
    CHMAX = max(ch for _, ch, _, _, _ in plan)
    assert sum(n * ch for n, ch, _, _, _ in plan) == NB

    @functools.partial(
        pl.kernel,
        out_type=jax.ShapeDtypeStruct((NC, NACC, F), jnp.float32),
        mesh=_mesh,
        compiler_params=pltpu.CompilerParams(use_tc_tiling_on_sc=False,
                                             needs_layout_passes=False),
        scratch_types=(
            [
                pltpu.VMEM_SHARED((NACC, F), jnp.float32),
                pltpu.VMEM((16, F), jnp.float32),
                pltpu.VMEM((CHMAX, EB), jnp.int32),
                pltpu.VMEM((CHMAX, EB), jnp.int32),
                pltpu.VMEM((CHMAX, EB), jnp.float32),
            ]
            + [pltpu.VMEM((EB, F), jnp.bfloat16)] * NG
            + [pltpu.VMEM((EB, F), jnp.float32)] * NS_
            + [pltpu.SemaphoreType.DMA] * (NG + NS_)
        ),
    )
    def _agg(hs0_hbm, hs1_hbm, src2_hbm, dst2_hbm, ew2_hbm, out_hbm,
             acc, zbuf, sidx, didx, ewb, *rest):
        brows = rest[:NG]
        frows = rest[NG:NG + NS_]
        gsem = rest[NG + NS_:2 * NG + NS_]
        ssem = rest[2 * NG + NS_:]
        cid = lax.axis_index("c")
        sid = lax.axis_index("s")

        for i in range(16):
            for j in range(F // 16):
                zbuf[i, pl.ds(j * 16, 16)] = jnp.zeros((16,), jnp.float32)

        @pl.loop(0, RPT // 16)
        def _zero(k):
            pltpu.sync_copy(zbuf, acc.at[pl.ds(sid * RPT + k * 16, 16), :])

        plsc.subcore_barrier()

        def _issue_gather(j, slot):
            @pl.when(cid == 0)
            def _g0():
                pltpu.async_copy(hs0_hbm.at[sidx.at[j]], brows[slot],
                                 gsem[slot])

            @pl.when(cid == 1)
            def _g1():
                pltpu.async_copy(hs1_hbm.at[sidx.at[j]], brows[slot],
                                 gsem[slot])

        def _wait_gather(j, slot):
            pltpu.make_async_copy(hs0_hbm.at[sidx.at[j]], brows[slot],
                                  gsem[slot]).wait()

        def _run_chunk(mrow, CH, G, S, L):
            pltpu.sync_copy(src2_hbm.at[pl.ds(mrow, CH), :], sidx.at[pl.ds(0, CH)])
            pltpu.sync_copy(dst2_hbm.at[pl.ds(mrow, CH), :], didx.at[pl.ds(0, CH)])
            pltpu.sync_copy(ew2_hbm.at[pl.ds(mrow, CH), :], ewb.at[pl.ds(0, CH)])

            for k in range(L):
                _issue_gather(k, k)

            step = G
            while step % S:
                step += G

            @pl.loop(0, CH, step=step)
            def _edge_batch(i):
                for b in range(step):
                    j = i + b
                    gs = b % G
                    ss = b % S
                    _wait_gather(j, gs)
                    jn = jnp.minimum(j + L, CH - 1)
                    _issue_gather(jn, (b + L) % G)

                    @pl.when(j >= S)
                    def _ws():
                        pltpu.make_async_copy(frows[ss],
                                              acc.at[didx.at[j]],
                                              ssem[ss]).wait()

                    for rb in range(EB // 16):
                        wv = ewb[j, pl.ds(rb * 16, 16)]
                        for rr in range(16):
                            r = rb * 16 + rr
                            w = wv[rr]
                            for c in range(F // 32):
                                v = brows[gs][r, pl.ds(c * 32, 32)]
                                pa, pb = plsc.unpack(
                                    v, format=plsc.PackFormat.INTERLEAVED,
                                    preferred_element_type=jnp.float32)
                                frows[ss][r, pl.ds(c * 32, 16)] = pa * w
                                frows[ss][r, pl.ds(c * 32 + 16, 16)] = pb * w

                    pltpu.async_copy(frows[ss], acc.at[didx.at[j]], ssem[ss],
                                     add=True)

            for k in range(L):
                pltpu.make_async_copy(hs0_hbm.at[sidx.at[CH - 1]], brows[k],
                                      gsem[k]).wait()
            for k in range(S):
                pltpu.make_async_copy(frows[k], acc.at[didx.at[CH - 1]],
                                      ssem[k]).wait()

        base = 0
        for (nch, CH, G, S, L) in plan:
            if nch == 1:
                _run_chunk(sid * NB + base, CH, G, S, L)
            else:
                @pl.loop(0, nch)
                def _chunk(ci, base=base, CH=CH, G=G, S=S, L=L):
                    _run_chunk(sid * NB + base + ci * CH, CH, G, S, L)
            base += nch * CH

        plsc.subcore_barrier()

        @pl.loop(0, RPT // EB)
        def _writeback(k):
            r0 = sid * RPT + k * EB
            pltpu.sync_copy(acc.at[pl.ds(r0, EB), :], frows[0])
            pltpu.sync_copy(frows[0], out_hbm.at[cid, pl.ds(r0, EB), :])

    return _agg


_agg_l1 = _make_agg(DH // 2, [(5, 42, 3, 2, 2), (1, 40, 2, 2, 1)], 3, 2)
_agg_l2 = _make_agg(DOUT // 2, [(1, 250, 5, 5, 3)], 5, 5)


def _ileave_bf16(h):
    n, f = h.shape
    h4 = h.reshape(n, f // 32, 2, 16)
    return jnp.stack([h4[:, :, 0, :], h4[:, :, 1, :]],
                     axis=-1).reshape(n, f).astype(jnp.bfloat16)


_R = 2000
_G = N // _R


def _tc1_body(d, x, w1, hsa, hsb):
    dis = lax.rsqrt(d[...] + 1.0)
    h = jnp.dot(x[...], w1[...], preferred_element_type=jnp.float32)
    hs = h * dis
    hsa[...] = hs[:, : DH // 2]
    hsb[...] = hs[:, DH // 2:]


def _tc1(d, x, w1):
    return pl.pallas_call(
        _tc1_body,
        grid=(_G,),
        in_specs=[
            pl.BlockSpec((_R, 1), lambda i: (i, 0)),
            pl.BlockSpec((_R, DIN), lambda i: (i, 0)),
            pl.BlockSpec((DIN, DH), lambda i: (0, 0)),
        ],
        out_specs=[
            pl.BlockSpec((_R, DH // 2), lambda i: (i, 0)),
            pl.BlockSpec((_R, DH // 2), lambda i: (i, 0)),
        ],
        out_shape=[
            jax.ShapeDtypeStruct((N, DH // 2), jnp.float32),
            jax.ShapeDtypeStruct((N, DH // 2), jnp.float32),
        ],
    )(d, x, w1)


def _tc2_body(d, a0, a1, hsa, hsb, b1, w2, o0, o1):
    dis = lax.rsqrt(d[...] + 1.0)
    agg = jnp.concatenate([a0[...] + hsa[...], a1[...] + hsb[...]], axis=1)
    t = jnp.maximum(dis * agg + b1[...], 0.0)
    hs2 = jnp.dot(t, w2[...], preferred_element_type=jnp.float32) * dis
    o0[...] = hs2[:, : DOUT // 2]
    o1[...] = hs2[:, DOUT // 2:]


def _tc2(d, a0, a1, hsa, hsb, b1, w2):
    return pl.pallas_call(
        _tc2_body,
        grid=(_G,),
        in_specs=[
            pl.BlockSpec((_R, 1), lambda i: (i, 0)),
            pl.BlockSpec((_R, DH // 2), lambda i: (i, 0)),
            pl.BlockSpec((_R, DH // 2), lambda i: (i, 0)),
            pl.BlockSpec((_R, DH // 2), lambda i: (i, 0)),
            pl.BlockSpec((_R, DH // 2), lambda i: (i, 0)),
            pl.BlockSpec((1, DH), lambda i: (0, 0)),
            pl.BlockSpec((DH, DOUT), lambda i: (0, 0)),
        ],
        out_specs=[
            pl.BlockSpec((_R, DOUT // 2), lambda i: (i, 0)),
            pl.BlockSpec((_R, DOUT // 2), lambda i: (i, 0)),
        ],
        out_shape=[
            jax.ShapeDtypeStruct((N, DOUT // 2), jnp.float32),
            jax.ShapeDtypeStruct((N, DOUT // 2), jnp.float32),
        ],
    )(d, a0, a1, hsa, hsb, b1, w2)


def _tc3_body(d, a0, a1, hsa, hsb, b2, out):
    dis = lax.rsqrt(d[...] + 1.0)
    o = jnp.concatenate([a0[...] + hsa[...], a1[...] + hsb[...]], axis=1)
    o = dis * o + b2[...]
    m = jnp.max(o, axis=1, keepdims=True)
    z = o - m
    out[...] = z - jnp.log(jnp.sum(jnp.exp(z), axis=1, keepdims=True))


def _tc3(d, a0, a1, hsa, hsb, b2):
    return pl.pallas_call(
        _tc3_body,
        grid=(_G,),
        in_specs=[
            pl.BlockSpec((_R, 1), lambda i: (i, 0)),
            pl.BlockSpec((_R, DOUT // 2), lambda i: (i, 0)),
            pl.BlockSpec((_R, DOUT // 2), lambda i: (i, 0)),
            pl.BlockSpec((_R, DOUT // 2), lambda i: (i, 0)),
            pl.BlockSpec((_R, DOUT // 2), lambda i: (i, 0)),
            pl.BlockSpec((1, DOUT), lambda i: (0, 0)),
        ],
        out_specs=pl.BlockSpec((_R, DOUT), lambda i: (i, 0)),
        out_shape=jax.ShapeDtypeStruct((N, DOUT), jnp.float32),
    )(d, a0, a1, hsa, hsb, b2)


def kernel(x, edge_index, edge_weight, W1, b1, W2, b2):
    src = edge_index[0]
    dst = edge_index[1]
    src2 = src.reshape(E // EB, EB)
    dst2 = dst.reshape(E // EB, EB)
    ew2 = edge_weight.reshape(E // EB, EB)

    deg = _deg_kernel(dst2, ew2)
    d = deg[:N].reshape(N, 1)

    hsa, hsb = _tc1(d, x, W1)

    agg1 = _agg_l1(_ileave_bf16(hsa), _ileave_bf16(hsb),
                   src2, dst2, ew2)
    hs2a, hs2b = _tc2(d, agg1[0, :N], agg1[1, :N],
                      hsa, hsb, b1.reshape(1, DH), W2)

    agg2 = _agg_l2(_ileave_bf16(hs2a), _ileave_bf16(hs2b),
                   src2, dst2, ew2)
    return _tc3(d, agg2[0, :N], agg2[1, :N],
                hs2a, hs2b, b2.reshape(1, DOUT))


# speedup vs baseline: 1.0530x; 1.0530x over previous
import functools

import jax
import jax.numpy as jnp
from jax import lax
from jax.experimental import pallas as pl
from jax.experimental.pallas import tpu as pltpu
from jax.experimental.pallas import tpu_sc as plsc

N = 10000
E = 320000
DIN = 128
DH = 256
DOUT = 64

NC = 2
NS = 16
NACC = 10240
RPT = NACC // NS
EB = 80
EPT = E // NS
NB = EPT // EB
NBW = NB

_mesh = plsc.VectorSubcoreMesh(core_axis_name="c", subcore_axis_name="s")


@functools.partial(
    pl.kernel,
    out_type=jax.ShapeDtypeStruct((NACC,), jnp.float32),
    mesh=_mesh,
    compiler_params=pltpu.CompilerParams(use_tc_tiling_on_sc=False),
    scratch_types=[
        pltpu.VMEM_SHARED((NACC,), jnp.float32),
        pltpu.VMEM((RPT,), jnp.float32),
        pltpu.VMEM((NBW, EB), jnp.int32),
        pltpu.VMEM((NBW, EB), jnp.float32),
        pltpu.SemaphoreType.DMA,
    ],
)
def _deg_kernel(dst2w_hbm, ew2w_hbm, out_hbm, acc, buf, didx, ewb, sem):
    cid = lax.axis_index("c")
    sid = lax.axis_index("s")
    for j in range(RPT // 16):
        buf[pl.ds(j * 16, 16)] = jnp.zeros((16,), jnp.float32)
    pltpu.sync_copy(buf, acc.at[pl.ds(sid * RPT, RPT)])

    pltpu.sync_copy(dst2w_hbm.at[pl.ds(sid * NBW, NBW), :], didx)
    pltpu.sync_copy(ew2w_hbm.at[pl.ds(sid * NBW, NBW), :], ewb)
    plsc.subcore_barrier()

    K = 5

    @pl.loop(0, NBW, step=K)
    def _edge_group(i):
        for k in range(K):
            pltpu.async_copy(ewb.at[i + k], acc.at[didx.at[i + k]], sem,
                             add=True)
        for k in range(K):
            pltpu.make_async_copy(ewb.at[i + k], acc.at[didx.at[i + k]],
                                  sem).wait()

    plsc.subcore_barrier()

    @pl.when(cid == 0)
    def _wb():
        pltpu.sync_copy(acc.at[pl.ds(sid * RPT, RPT)], buf)
        pltpu.sync_copy(buf, out_hbm.at[pl.ds(sid * RPT, RPT)])


def _make_agg(F, plan, NSLOT):
    CHMAX = max(ch for _, ch, _, _ in plan)
    assert sum(n * ch for n, ch, _, _ in plan) == NB

    @functools.partial(
        pl.kernel,
        out_type=jax.ShapeDtypeStruct((NC, NACC, F), jnp.float32),
        mesh=_mesh,
        compiler_params=pltpu.CompilerParams(use_tc_tiling_on_sc=False),
        scratch_types=(
            [
                pltpu.VMEM_SHARED((NACC, F), jnp.float32),
                pltpu.VMEM((16, F), jnp.float32),
                pltpu.VMEM((CHMAX, EB), jnp.int32),
                pltpu.VMEM((CHMAX, EB), jnp.int32),
                pltpu.VMEM((CHMAX, EB), jnp.float32),
            ]
            + [pltpu.VMEM((EB, F), jnp.float32)] * NSLOT
            + [pltpu.SemaphoreType.DMA] * (2 * NSLOT)
        ),
    )
    def _agg(hs0_hbm, hs1_hbm, src2_hbm, dst2_hbm, ew2_hbm, out_hbm,
             acc, zbuf, sidx, didx, ewb, *rest):
        rows = rest[:NSLOT]
        gsem = rest[NSLOT:2 * NSLOT]
        ssem = rest[2 * NSLOT:3 * NSLOT]
        cid = lax.axis_index("c")
        sid = lax.axis_index("s")

        for i in range(16):
            for j in range(F // 16):
                zbuf[i, pl.ds(j * 16, 16)] = jnp.zeros((16,), jnp.float32)

        @pl.loop(0, RPT // 16)
        def _zero(k):
            pltpu.sync_copy(zbuf, acc.at[pl.ds(sid * RPT + k * 16, 16), :])

        plsc.subcore_barrier()

        def _issue_gather(j, slot):
            @pl.when(cid == 0)
            def _g0():
                pltpu.async_copy(hs0_hbm.at[sidx.at[j]], rows[slot],
                                 gsem[slot])

            @pl.when(cid == 1)
            def _g1():
                pltpu.async_copy(hs1_hbm.at[sidx.at[j]], rows[slot],
                                 gsem[slot])

        def _wait_gather(j, slot):
            pltpu.make_async_copy(hs0_hbm.at[sidx.at[j]], rows[slot],
                                  gsem[slot]).wait()

        def _run_chunk(mrow, CH, nslot, L):
            pltpu.sync_copy(src2_hbm.at[pl.ds(mrow, CH), :], sidx.at[pl.ds(0, CH)])
            pltpu.sync_copy(dst2_hbm.at[pl.ds(mrow, CH), :], didx.at[pl.ds(0, CH)])
            pltpu.sync_copy(ew2_hbm.at[pl.ds(mrow, CH), :], ewb.at[pl.ds(0, CH)])

            for k in range(L):
                _issue_gather(k, k)

            @pl.loop(0, CH, step=nslot)
            def _edge_batch(i):
                for b in range(nslot):
                    j = i + b
                    _wait_gather(j, b)
                    nslot_b = (b + L) % nslot

                    @pl.when(j >= nslot - L)
                    def _ws():
                        pltpu.make_async_copy(rows[nslot_b],
                                              acc.at[didx.at[j]],
                                              ssem[nslot_b]).wait()

                    jn = jnp.minimum(j + L, CH - 1)
                    _issue_gather(jn, nslot_b)

                    for rb in range(EB // 16):
                        wv = ewb[j, pl.ds(rb * 16, 16)]
                        for rr in range(16):
                            r = rb * 16 + rr
                            w = wv[rr]
                            for c in range(F // 16):
                                rows[b][r, pl.ds(c * 16, 16)] = (
                                    rows[b][r, pl.ds(c * 16, 16)] * w)

                    pltpu.async_copy(rows[b], acc.at[didx.at[j]], ssem[b],
                                     add=True)

            for k in range(L):
                pltpu.make_async_copy(hs0_hbm.at[sidx.at[CH - 1]], rows[k],
                                      gsem[k]).wait()
            for k in range(nslot - L):
                slot = (L + k) % nslot
                pltpu.make_async_copy(rows[slot], acc.at[didx.at[CH - 1]],
                                      ssem[slot]).wait()

        base = 0
        for (nch, CH, nslot, L) in plan:
            if nch == 1:
                _run_chunk(sid * NB + base, CH, nslot, L)
            else:
                @pl.loop(0, nch)
                def _chunk(ci, base=base, CH=CH, nslot=nslot, L=L):
                    _run_chunk(sid * NB + base + ci * CH, CH, nslot, L)
            base += nch * CH

        plsc.subcore_barrier()

        @pl.loop(0, RPT // EB)
        def _writeback(k):
            r0 = sid * RPT + k * EB
            pltpu.sync_copy(acc.at[pl.ds(r0, EB), :], rows[0])
            pltpu.sync_copy(rows[0], out_hbm.at[cid, pl.ds(r0, EB), :])

    return _agg


_agg_l1 = _make_agg(DH // 2, [(10, 24, 4, 2), (1, 10, 2, 1)], 4)
_agg_l2 = _make_agg(DOUT // 2, [(1, 250, 5, 3)], 5)


_R = 2000
_G = N // _R


def _tc1_body(d, x, w1, hsa, hsb):
    dis = lax.rsqrt(d[...] + 1.0)
    h = jnp.dot(x[...], w1[...], preferred_element_type=jnp.float32)
    hs = h * dis
    hsa[...] = hs[:, : DH // 2]
    hsb[...] = hs[:, DH // 2:]


def _tc1(d, x, w1):
    return pl.pallas_call(
        _tc1_body,
        grid=(_G,),
        in_specs=[
            pl.BlockSpec((_R, 1), lambda i: (i, 0)),
            pl.BlockSpec((_R, DIN), lambda i: (i, 0)),
            pl.BlockSpec((DIN, DH), lambda i: (0, 0)),
        ],
        out_specs=[
            pl.BlockSpec((_R, DH // 2), lambda i: (i, 0)),
            pl.BlockSpec((_R, DH // 2), lambda i: (i, 0)),
        ],
        out_shape=[
            jax.ShapeDtypeStruct((N, DH // 2), jnp.float32),
            jax.ShapeDtypeStruct((N, DH // 2), jnp.float32),
        ],
    )(d, x, w1)


def _tc2_body(d, a0, a1, hsa, hsb, b1, w2, o0, o1):
    dis = lax.rsqrt(d[...] + 1.0)
    agg = jnp.concatenate([a0[...] + hsa[...], a1[...] + hsb[...]], axis=1)
    t = jnp.maximum(dis * agg + b1[...], 0.0)
    hs2 = jnp.dot(t, w2[...], preferred_element_type=jnp.float32) * dis
    o0[...] = hs2[:, : DOUT // 2]
    o1[...] = hs2[:, DOUT // 2:]


def _tc2(d, a0, a1, hsa, hsb, b1, w2):
    return pl.pallas_call(
        _tc2_body,
        grid=(_G,),
        in_specs=[
            pl.BlockSpec((_R, 1), lambda i: (i, 0)),
            pl.BlockSpec((_R, DH // 2), lambda i: (i, 0)),
            pl.BlockSpec((_R, DH // 2), lambda i: (i, 0)),
            pl.BlockSpec((_R, DH // 2), lambda i: (i, 0)),
            pl.BlockSpec((_R, DH // 2), lambda i: (i, 0)),
            pl.BlockSpec((1, DH), lambda i: (0, 0)),
            pl.BlockSpec((DH, DOUT), lambda i: (0, 0)),
        ],
        out_specs=[
            pl.BlockSpec((_R, DOUT // 2), lambda i: (i, 0)),
            pl.BlockSpec((_R, DOUT // 2), lambda i: (i, 0)),
        ],
        out_shape=[
            jax.ShapeDtypeStruct((N, DOUT // 2), jnp.float32),
            jax.ShapeDtypeStruct((N, DOUT // 2), jnp.float32),
        ],
    )(d, a0, a1, hsa, hsb, b1, w2)


def _tc3_body(d, a0, a1, hsa, hsb, b2, out):
    dis = lax.rsqrt(d[...] + 1.0)
    o = jnp.concatenate([a0[...] + hsa[...], a1[...] + hsb[...]], axis=1)
    o = dis * o + b2[...]
    m = jnp.max(o, axis=1, keepdims=True)
    z = o - m
    out[...] = z - jnp.log(jnp.sum(jnp.exp(z), axis=1, keepdims=True))


def _tc3(d, a0, a1, hsa, hsb, b2):
    return pl.pallas_call(
        _tc3_body,
        grid=(_G,),
        in_specs=[
            pl.BlockSpec((_R, 1), lambda i: (i, 0)),
            pl.BlockSpec((_R, DOUT // 2), lambda i: (i, 0)),
            pl.BlockSpec((_R, DOUT // 2), lambda i: (i, 0)),
            pl.BlockSpec((_R, DOUT // 2), lambda i: (i, 0)),
            pl.BlockSpec((_R, DOUT // 2), lambda i: (i, 0)),
            pl.BlockSpec((1, DOUT), lambda i: (0, 0)),
        ],
        out_specs=pl.BlockSpec((_R, DOUT), lambda i: (i, 0)),
        out_shape=jax.ShapeDtypeStruct((N, DOUT), jnp.float32),
    )(d, a0, a1, hsa, hsb, b2)


def kernel(x, edge_index, edge_weight, W1, b1, W2, b2):
    src = edge_index[0]
    dst = edge_index[1]
    src2 = src.reshape(E // EB, EB)
    dst2 = dst.reshape(E // EB, EB)
    ew2 = edge_weight.reshape(E // EB, EB)

    deg = _deg_kernel(dst2, ew2)
    d = deg[:N].reshape(N, 1)

    hsa, hsb = _tc1(d, x, W1)

    agg1 = _agg_l1(hsa, hsb, src2, dst2, ew2)
    hs2a, hs2b = _tc2(d, agg1[0, :N], agg1[1, :N],
                      hsa, hsb, b1.reshape(1, DH), W2)

    agg2 = _agg_l2(hs2a, hs2b, src2, dst2, ew2)
    return _tc3(d, agg2[0, :N], agg2[1, :N],
                hs2a, hs2b, b2.reshape(1, DOUT))


# speedup vs baseline: 1.0903x; 1.0355x over previous
import functools

import jax
import jax.numpy as jnp
from jax import lax
from jax.experimental import pallas as pl
from jax.experimental.pallas import tpu as pltpu
from jax.experimental.pallas import tpu_sc as plsc

N = 10000
E = 320000
DIN = 128
DH = 256
DOUT = 64

NC = 2
NS = 16
NACC = 10240
RPT = NACC // NS
EB = 80
EPT = E // NS
NB = EPT // EB
NBW = NB

_mesh = plsc.VectorSubcoreMesh(core_axis_name="c", subcore_axis_name="s")


@functools.partial(
    pl.kernel,
    out_type=jax.ShapeDtypeStruct((NACC,), jnp.float32),
    mesh=_mesh,
    compiler_params=pltpu.CompilerParams(use_tc_tiling_on_sc=False),
    scratch_types=[
        pltpu.VMEM_SHARED((NACC,), jnp.float32),
        pltpu.VMEM((RPT,), jnp.float32),
        pltpu.VMEM((NBW, EB), jnp.int32),
        pltpu.VMEM((NBW, EB), jnp.float32),
        pltpu.SemaphoreType.DMA,
    ],
)
def _deg_kernel(dst2w_hbm, ew2w_hbm, out_hbm, acc, buf, didx, ewb, sem):
    cid = lax.axis_index("c")
    sid = lax.axis_index("s")
    for j in range(RPT // 16):
        buf[pl.ds(j * 16, 16)] = jnp.zeros((16,), jnp.float32)
    pltpu.sync_copy(buf, acc.at[pl.ds(sid * RPT, RPT)])

    pltpu.sync_copy(dst2w_hbm.at[pl.ds(sid * NBW, NBW), :], didx)
    pltpu.sync_copy(ew2w_hbm.at[pl.ds(sid * NBW, NBW), :], ewb)
    plsc.subcore_barrier()

    K = 5

    @pl.loop(0, NBW, step=K)
    def _edge_group(i):
        for k in range(K):
            pltpu.async_copy(ewb.at[i + k], acc.at[didx.at[i + k]], sem,
                             add=True)
        for k in range(K):
            pltpu.make_async_copy(ewb.at[i + k], acc.at[didx.at[i + k]],
                                  sem).wait()

    plsc.subcore_barrier()

    @pl.when(cid == 0)
    def _wb():
        pltpu.sync_copy(acc.at[pl.ds(sid * RPT, RPT)], buf)
        pltpu.sync_copy(buf, out_hbm.at[pl.ds(sid * RPT, RPT)])


def _make_agg(F, plan, NSLOT):
    CHMAX = max(ch for _, ch, _, _ in plan)
    assert sum(n * ch for n, ch, _, _ in plan) == NB

    @functools.partial(
        pl.kernel,
        out_type=jax.ShapeDtypeStruct((NC, NACC, F), jnp.float32),
        mesh=_mesh,
        compiler_params=pltpu.CompilerParams(use_tc_tiling_on_sc=False),
        scratch_types=(
            [
                pltpu.VMEM_SHARED((NACC, F), jnp.float32),
                pltpu.VMEM((16, F), jnp.float32),
                pltpu.VMEM((CHMAX, EB), jnp.int32),
                pltpu.VMEM((CHMAX, EB), jnp.int32),
                pltpu.VMEM((CHMAX, EB), jnp.float32),
            ]
            + [pltpu.VMEM((EB, F), jnp.float32)] * NSLOT
            + [pltpu.SemaphoreType.DMA] * (2 * NSLOT)
        ),
    )
    def _agg(hs0_hbm, hs1_hbm, src2_hbm, dst2_hbm, ew2_hbm, out_hbm,
             acc, zbuf, sidx, didx, ewb, *rest):
        rows = rest[:NSLOT]
        gsem = rest[NSLOT:2 * NSLOT]
        ssem = rest[2 * NSLOT:3 * NSLOT]
        cid = lax.axis_index("c")
        sid = lax.axis_index("s")

        for i in range(16):
            for j in range(F // 16):
                zbuf[i, pl.ds(j * 16, 16)] = jnp.zeros((16,), jnp.float32)

        @pl.loop(0, RPT // 16)
        def _zero(k):
            pltpu.sync_copy(zbuf, acc.at[pl.ds(sid * RPT + k * 16, 16), :])

        plsc.subcore_barrier()

        def _issue_gather(j, slot):
            @pl.when(cid == 0)
            def _g0():
                pltpu.async_copy(hs0_hbm.at[sidx.at[j]], rows[slot],
                                 gsem[slot])

            @pl.when(cid == 1)
            def _g1():
                pltpu.async_copy(hs1_hbm.at[sidx.at[j]], rows[slot],
                                 gsem[slot])

        def _wait_gather(j, slot):
            pltpu.make_async_copy(hs0_hbm.at[sidx.at[j]], rows[slot],
                                  gsem[slot]).wait()

        def _run_chunk(mrow, CH, nslot, L):
            pltpu.sync_copy(src2_hbm.at[pl.ds(mrow, CH), :], sidx.at[pl.ds(0, CH)])
            pltpu.sync_copy(dst2_hbm.at[pl.ds(mrow, CH), :], didx.at[pl.ds(0, CH)])
            pltpu.sync_copy(ew2_hbm.at[pl.ds(mrow, CH), :], ewb.at[pl.ds(0, CH)])

            for k in range(L):
                _issue_gather(k, k)

            @pl.loop(0, CH, step=nslot)
            def _edge_batch(i):
                for b in range(nslot):
                    j = i + b
                    _wait_gather(j, b)
                    nslot_b = (b + L) % nslot

                    @pl.when(j >= nslot - L)
                    def _ws():
                        pltpu.make_async_copy(rows[nslot_b],
                                              acc.at[didx.at[j]],
                                              ssem[nslot_b]).wait()

                    jn = jnp.minimum(j + L, CH - 1)
                    _issue_gather(jn, nslot_b)

                    for rb in range(EB // 16):
                        wv = ewb[j, pl.ds(rb * 16, 16)]
                        for rr in range(16):
                            r = rb * 16 + rr
                            w = wv[rr]
                            for c in range(F // 16):
                                rows[b][r, pl.ds(c * 16, 16)] = (
                                    rows[b][r, pl.ds(c * 16, 16)] * w)

                    pltpu.async_copy(rows[b], acc.at[didx.at[j]], ssem[b],
                                     add=True)

            for k in range(L):
                pltpu.make_async_copy(hs0_hbm.at[sidx.at[CH - 1]], rows[k],
                                      gsem[k]).wait()
            for k in range(nslot - L):
                slot = (L + k) % nslot
                pltpu.make_async_copy(rows[slot], acc.at[didx.at[CH - 1]],
                                      ssem[slot]).wait()

        base = 0
        for (nch, CH, nslot, L) in plan:
            if nch == 1:
                _run_chunk(sid * NB + base, CH, nslot, L)
            else:
                @pl.loop(0, nch)
                def _chunk(ci, base=base, CH=CH, nslot=nslot, L=L):
                    _run_chunk(sid * NB + base + ci * CH, CH, nslot, L)
            base += nch * CH

        plsc.subcore_barrier()

        @pl.loop(0, RPT // EB)
        def _writeback(k):
            r0 = sid * RPT + k * EB
            pltpu.sync_copy(acc.at[pl.ds(r0, EB), :], rows[0])
            pltpu.sync_copy(rows[0], out_hbm.at[cid, pl.ds(r0, EB), :])

    return _agg


_agg_l1 = _make_agg(DH // 2, [(5, 48, 3, 1), (1, 10, 2, 1)], 3)
_agg_l2 = _make_agg(DOUT // 2, [(1, 250, 5, 3)], 5)


_R = 2000
_G = N // _R


def _tc1_body(d, x, w1, hsa, hsb):
    dis = lax.rsqrt(d[...] + 1.0)
    h = jnp.dot(x[...], w1[...], preferred_element_type=jnp.float32)
    hs = h * dis
    hsa[...] = hs[:, : DH // 2]
    hsb[...] = hs[:, DH // 2:]


def _tc1(d, x, w1):
    return pl.pallas_call(
        _tc1_body,
        grid=(_G,),
        in_specs=[
            pl.BlockSpec((_R, 1), lambda i: (i, 0)),
            pl.BlockSpec((_R, DIN), lambda i: (i, 0)),
            pl.BlockSpec((DIN, DH), lambda i: (0, 0)),
        ],
        out_specs=[
            pl.BlockSpec((_R, DH // 2), lambda i: (i, 0)),
            pl.BlockSpec((_R, DH // 2), lambda i: (i, 0)),
        ],
        out_shape=[
            jax.ShapeDtypeStruct((N, DH // 2), jnp.float32),
            jax.ShapeDtypeStruct((N, DH // 2), jnp.float32),
        ],
    )(d, x, w1)


def _tc2_body(d, a0, a1, hsa, hsb, b1, w2, o0, o1):
    dis = lax.rsqrt(d[...] + 1.0)
    agg = jnp.concatenate([a0[...] + hsa[...], a1[...] + hsb[...]], axis=1)
    t = jnp.maximum(dis * agg + b1[...], 0.0)
    hs2 = jnp.dot(t, w2[...], preferred_element_type=jnp.float32) * dis
    o0[...] = hs2[:, : DOUT // 2]
    o1[...] = hs2[:, DOUT // 2:]


def _tc2(d, a0, a1, hsa, hsb, b1, w2):
    return pl.pallas_call(
        _tc2_body,
        grid=(_G,),
        in_specs=[
            pl.BlockSpec((_R, 1), lambda i: (i, 0)),
            pl.BlockSpec((_R, DH // 2), lambda i: (i, 0)),
            pl.BlockSpec((_R, DH // 2), lambda i: (i, 0)),
            pl.BlockSpec((_R, DH // 2), lambda i: (i, 0)),
            pl.BlockSpec((_R, DH // 2), lambda i: (i, 0)),
            pl.BlockSpec((1, DH), lambda i: (0, 0)),
            pl.BlockSpec((DH, DOUT), lambda i: (0, 0)),
        ],
        out_specs=[
            pl.BlockSpec((_R, DOUT // 2), lambda i: (i, 0)),
            pl.BlockSpec((_R, DOUT // 2), lambda i: (i, 0)),
        ],
        out_shape=[
            jax.ShapeDtypeStruct((N, DOUT // 2), jnp.float32),
            jax.ShapeDtypeStruct((N, DOUT // 2), jnp.float32),
        ],
    )(d, a0, a1, hsa, hsb, b1, w2)


def _tc3_body(d, a0, a1, hsa, hsb, b2, out):
    dis = lax.rsqrt(d[...] + 1.0)
    o = jnp.concatenate([a0[...] + hsa[...], a1[...] + hsb[...]], axis=1)
    o = dis * o + b2[...]
    m = jnp.max(o, axis=1, keepdims=True)
    z = o - m
    out[...] = z - jnp.log(jnp.sum(jnp.exp(z), axis=1, keepdims=True))


def _tc3(d, a0, a1, hsa, hsb, b2):
    return pl.pallas_call(
        _tc3_body,
        grid=(_G,),
        in_specs=[
            pl.BlockSpec((_R, 1), lambda i: (i, 0)),
            pl.BlockSpec((_R, DOUT // 2), lambda i: (i, 0)),
            pl.BlockSpec((_R, DOUT // 2), lambda i: (i, 0)),
            pl.BlockSpec((_R, DOUT // 2), lambda i: (i, 0)),
            pl.BlockSpec((_R, DOUT // 2), lambda i: (i, 0)),
            pl.BlockSpec((1, DOUT), lambda i: (0, 0)),
        ],
        out_specs=pl.BlockSpec((_R, DOUT), lambda i: (i, 0)),
        out_shape=jax.ShapeDtypeStruct((N, DOUT), jnp.float32),
    )(d, a0, a1, hsa, hsb, b2)


def kernel(x, edge_index, edge_weight, W1, b1, W2, b2):
    src = edge_index[0]
    dst = edge_index[1]
    src2 = src.reshape(E // EB, EB)
    dst2 = dst.reshape(E // EB, EB)
    ew2 = edge_weight.reshape(E // EB, EB)

    deg = _deg_kernel(dst2, ew2)
    d = deg[:N].reshape(N, 1)

    hsa, hsb = _tc1(d, x, W1)

    agg1 = _agg_l1(hsa, hsb, src2, dst2, ew2)
    hs2a, hs2b = _tc2(d, agg1[0, :N], agg1[1, :N],
                      hsa, hsb, b1.reshape(1, DH), W2)

    agg2 = _agg_l2(hs2a, hs2b, src2, dst2, ew2)
    return _tc3(d, agg2[0, :N], agg2[1, :N],
                hs2a, hs2b, b2.reshape(1, DOUT))


# speedup vs baseline: 1.1844x; 1.0863x over previous
import functools

import jax
import jax.numpy as jnp
from jax import lax
from jax.experimental import pallas as pl
from jax.experimental.pallas import tpu as pltpu
from jax.experimental.pallas import tpu_sc as plsc

N = 10000
E = 320000
DIN = 128
DH = 256
DOUT = 64

NC = 2
NS = 16
NACC = 10240
RPT = NACC // NS
EB = 80
EPT = E // NS
NB = EPT // EB
NBW = NB

_mesh = plsc.VectorSubcoreMesh(core_axis_name="c", subcore_axis_name="s")


@functools.partial(
    pl.kernel,
    out_type=jax.ShapeDtypeStruct((NACC,), jnp.float32),
    mesh=_mesh,
    compiler_params=pltpu.CompilerParams(use_tc_tiling_on_sc=False),
    scratch_types=[
        pltpu.VMEM_SHARED((NACC,), jnp.float32),
        pltpu.VMEM((RPT,), jnp.float32),
        pltpu.VMEM((NBW, EB), jnp.int32),
        pltpu.VMEM((NBW, EB), jnp.float32),
        pltpu.SemaphoreType.DMA,
    ],
)
def _deg_kernel(dst2w_hbm, ew2w_hbm, out_hbm, acc, buf, didx, ewb, sem):
    cid = lax.axis_index("c")
    sid = lax.axis_index("s")
    for j in range(RPT // 16):
        buf[pl.ds(j * 16, 16)] = jnp.zeros((16,), jnp.float32)
    pltpu.sync_copy(buf, acc.at[pl.ds(sid * RPT, RPT)])

    pltpu.sync_copy(dst2w_hbm.at[pl.ds(sid * NBW, NBW), :], didx)
    pltpu.sync_copy(ew2w_hbm.at[pl.ds(sid * NBW, NBW), :], ewb)
    plsc.subcore_barrier()

    K = 5

    @pl.loop(0, NBW, step=K)
    def _edge_group(i):
        for k in range(K):
            pltpu.async_copy(ewb.at[i + k], acc.at[didx.at[i + k]], sem,
                             add=True)
        for k in range(K):
            pltpu.make_async_copy(ewb.at[i + k], acc.at[didx.at[i + k]],
                                  sem).wait()

    plsc.subcore_barrier()

    @pl.when(cid == 0)
    def _wb():
        pltpu.sync_copy(acc.at[pl.ds(sid * RPT, RPT)], buf)
        pltpu.sync_copy(buf, out_hbm.at[pl.ds(sid * RPT, RPT)])


def _make_agg(F, plan, NSLOT):
    CHMAX = max(ch for _, ch, _, _ in plan)
    assert sum(n * ch for n, ch, _, _ in plan) == NB

    @functools.partial(
        pl.kernel,
        out_type=jax.ShapeDtypeStruct((NC, NACC, F), jnp.float32),
        mesh=_mesh,
        compiler_params=pltpu.CompilerParams(use_tc_tiling_on_sc=False),
        scratch_types=(
            [
                pltpu.VMEM_SHARED((NACC, F), jnp.float32),
                pltpu.VMEM((16, F), jnp.float32),
                pltpu.VMEM((CHMAX, EB), jnp.int32),
                pltpu.VMEM((CHMAX, EB), jnp.int32),
                pltpu.VMEM((CHMAX, EB), jnp.float32),
            ]
            + [pltpu.VMEM((EB, F), jnp.float32)] * NSLOT
            + [pltpu.SemaphoreType.DMA] * (2 * NSLOT)
        ),
    )
    def _agg(hs0_hbm, hs1_hbm, src2_hbm, dst2_hbm, ew2_hbm, out_hbm,
             acc, zbuf, sidx, didx, ewb, *rest):
        rows = rest[:NSLOT]
        gsem = rest[NSLOT:2 * NSLOT]
        ssem = rest[2 * NSLOT:3 * NSLOT]
        cid = lax.axis_index("c")
        sid = lax.axis_index("s")

        for i in range(16):
            for j in range(F // 16):
                zbuf[i, pl.ds(j * 16, 16)] = jnp.zeros((16,), jnp.float32)

        @pl.loop(0, RPT // 16)
        def _zero(k):
            pltpu.sync_copy(zbuf, acc.at[pl.ds(sid * RPT + k * 16, 16), :])

        plsc.subcore_barrier()

        def _issue_gather(j, slot):
            @pl.when(cid == 0)
            def _g0():
                pltpu.async_copy(hs0_hbm.at[sidx.at[j]], rows[slot],
                                 gsem[slot])

            @pl.when(cid == 1)
            def _g1():
                pltpu.async_copy(hs1_hbm.at[sidx.at[j]], rows[slot],
                                 gsem[slot])

        def _wait_gather(j, slot):
            pltpu.make_async_copy(hs0_hbm.at[sidx.at[j]], rows[slot],
                                  gsem[slot]).wait()

        def _run_chunk(mrow, CH, nslot, L):
            pltpu.sync_copy(src2_hbm.at[pl.ds(mrow, CH), :], sidx.at[pl.ds(0, CH)])
            pltpu.sync_copy(dst2_hbm.at[pl.ds(mrow, CH), :], didx.at[pl.ds(0, CH)])
            pltpu.sync_copy(ew2_hbm.at[pl.ds(mrow, CH), :], ewb.at[pl.ds(0, CH)])

            for k in range(L):
                _issue_gather(k, k)

            @pl.loop(0, CH, step=nslot)
            def _edge_batch(i):
                for b in range(nslot):
                    j = i + b
                    _wait_gather(j, b)
                    nslot_b = (b + L) % nslot

                    @pl.when(j >= nslot - L)
                    def _ws():
                        pltpu.make_async_copy(rows[nslot_b],
                                              acc.at[didx.at[j]],
                                              ssem[nslot_b]).wait()

                    jn = jnp.minimum(j + L, CH - 1)
                    _issue_gather(jn, nslot_b)

                    for rb in range(EB // 16):
                        wv = ewb[j, pl.ds(rb * 16, 16)]
                        for rr in range(16):
                            r = rb * 16 + rr
                            w = wv[rr]
                            for c in range(F // 16):
                                rows[b][r, pl.ds(c * 16, 16)] = (
                                    rows[b][r, pl.ds(c * 16, 16)] * w)

                    pltpu.async_copy(rows[b], acc.at[didx.at[j]], ssem[b],
                                     add=True)

            for k in range(L):
                pltpu.make_async_copy(hs0_hbm.at[sidx.at[CH - 1]], rows[k],
                                      gsem[k]).wait()
            for k in range(nslot - L):
                slot = (L + k) % nslot
                pltpu.make_async_copy(rows[slot], acc.at[didx.at[CH - 1]],
                                      ssem[slot]).wait()

        base = 0
        for (nch, CH, nslot, L) in plan:
            if nch == 1:
                _run_chunk(sid * NB + base, CH, nslot, L)
            else:
                @pl.loop(0, nch)
                def _chunk(ci, base=base, CH=CH, nslot=nslot, L=L):
                    _run_chunk(sid * NB + base + ci * CH, CH, nslot, L)
            base += nch * CH

        plsc.subcore_barrier()

        @pl.loop(0, RPT // EB)
        def _writeback(k):
            r0 = sid * RPT + k * EB
            pltpu.sync_copy(acc.at[pl.ds(r0, EB), :], rows[0])
            pltpu.sync_copy(rows[0], out_hbm.at[cid, pl.ds(r0, EB), :])

    return _agg


_agg_l1 = _make_agg(DH // 2, [(5, 50, 2, 1)], 2)
_agg_l2 = _make_agg(DOUT // 2, [(1, 250, 5, 3)], 5)


_R = 2000
_G = N // _R


def _tc1_body(d, x, w1, hsa, hsb):
    dis = lax.rsqrt(d[...] + 1.0)
    h = jnp.dot(x[...], w1[...], preferred_element_type=jnp.float32)
    hs = h * dis
    hsa[...] = hs[:, : DH // 2]
    hsb[...] = hs[:, DH // 2:]


def _tc1(d, x, w1):
    return pl.pallas_call(
        _tc1_body,
        grid=(_G,),
        in_specs=[
            pl.BlockSpec((_R, 1), lambda i: (i, 0)),
            pl.BlockSpec((_R, DIN), lambda i: (i, 0)),
            pl.BlockSpec((DIN, DH), lambda i: (0, 0)),
        ],
        out_specs=[
            pl.BlockSpec((_R, DH // 2), lambda i: (i, 0)),
            pl.BlockSpec((_R, DH // 2), lambda i: (i, 0)),
        ],
        out_shape=[
            jax.ShapeDtypeStruct((N, DH // 2), jnp.float32),
            jax.ShapeDtypeStruct((N, DH // 2), jnp.float32),
        ],
    )(d, x, w1)


def _tc2_body(d, a0, a1, hsa, hsb, b1, w2, o0, o1):
    dis = lax.rsqrt(d[...] + 1.0)
    agg = jnp.concatenate([a0[...] + hsa[...], a1[...] + hsb[...]], axis=1)
    t = jnp.maximum(dis * agg + b1[...], 0.0)
    hs2 = jnp.dot(t, w2[...], preferred_element_type=jnp.float32) * dis
    o0[...] = hs2[:, : DOUT // 2]
    o1[...] = hs2[:, DOUT // 2:]


def _tc2(d, a0, a1, hsa, hsb, b1, w2):
    return pl.pallas_call(
        _tc2_body,
        grid=(_G,),
        in_specs=[
            pl.BlockSpec((_R, 1), lambda i: (i, 0)),
            pl.BlockSpec((_R, DH // 2), lambda i: (i, 0)),
            pl.BlockSpec((_R, DH // 2), lambda i: (i, 0)),
            pl.BlockSpec((_R, DH // 2), lambda i: (i, 0)),
            pl.BlockSpec((_R, DH // 2), lambda i: (i, 0)),
            pl.BlockSpec((1, DH), lambda i: (0, 0)),
            pl.BlockSpec((DH, DOUT), lambda i: (0, 0)),
        ],
        out_specs=[
            pl.BlockSpec((_R, DOUT // 2), lambda i: (i, 0)),
            pl.BlockSpec((_R, DOUT // 2), lambda i: (i, 0)),
        ],
        out_shape=[
            jax.ShapeDtypeStruct((N, DOUT // 2), jnp.float32),
            jax.ShapeDtypeStruct((N, DOUT // 2), jnp.float32),
        ],
    )(d, a0, a1, hsa, hsb, b1, w2)


def _tc3_body(d, a0, a1, hsa, hsb, b2, out):
    dis = lax.rsqrt(d[...] + 1.0)
    o = jnp.concatenate([a0[...] + hsa[...], a1[...] + hsb[...]], axis=1)
    o = dis * o + b2[...]
    m = jnp.max(o, axis=1, keepdims=True)
    z = o - m
    out[...] = z - jnp.log(jnp.sum(jnp.exp(z), axis=1, keepdims=True))


def _tc3(d, a0, a1, hsa, hsb, b2):
    return pl.pallas_call(
        _tc3_body,
        grid=(_G,),
        in_specs=[
            pl.BlockSpec((_R, 1), lambda i: (i, 0)),
            pl.BlockSpec((_R, DOUT // 2), lambda i: (i, 0)),
            pl.BlockSpec((_R, DOUT // 2), lambda i: (i, 0)),
            pl.BlockSpec((_R, DOUT // 2), lambda i: (i, 0)),
            pl.BlockSpec((_R, DOUT // 2), lambda i: (i, 0)),
            pl.BlockSpec((1, DOUT), lambda i: (0, 0)),
        ],
        out_specs=pl.BlockSpec((_R, DOUT), lambda i: (i, 0)),
        out_shape=jax.ShapeDtypeStruct((N, DOUT), jnp.float32),
    )(d, a0, a1, hsa, hsb, b2)


def kernel(x, edge_index, edge_weight, W1, b1, W2, b2):
    src = edge_index[0]
    dst = edge_index[1]
    src2 = src.reshape(E // EB, EB)
    dst2 = dst.reshape(E // EB, EB)
    ew2 = edge_weight.reshape(E // EB, EB)

    deg = _deg_kernel(dst2, ew2)
    d = deg[:N].reshape(N, 1)

    hsa, hsb = _tc1(d, x, W1)

    agg1 = _agg_l1(hsa, hsb, src2, dst2, ew2)
    hs2a, hs2b = _tc2(d, agg1[0, :N], agg1[1, :N],
                      hsa, hsb, b1.reshape(1, DH), W2)

    agg2 = _agg_l2(hs2a, hs2b, src2, dst2, ew2)
    return _tc3(d, agg2[0, :N], agg2[1, :N],
                hs2a, hs2b, b2.reshape(1, DOUT))


# speedup vs baseline: 1.1917x; 1.0061x over previous
import functools

import jax
import jax.numpy as jnp
from jax import lax
from jax.experimental import pallas as pl
from jax.experimental.pallas import tpu as pltpu
from jax.experimental.pallas import tpu_sc as plsc

N = 10000
E = 320000
DIN = 128
DH = 256
DOUT = 64

NC = 2
NS = 16
NACC = 10240
RPT = NACC // NS
EB = 80
EPT = E // NS
NB = EPT // EB
NBW = NB

_mesh = plsc.VectorSubcoreMesh(core_axis_name="c", subcore_axis_name="s")


@functools.partial(
    pl.kernel,
    out_type=jax.ShapeDtypeStruct((NACC,), jnp.float32),
    mesh=_mesh,
    compiler_params=pltpu.CompilerParams(use_tc_tiling_on_sc=False),
    scratch_types=[
        pltpu.VMEM_SHARED((NACC,), jnp.float32),
        pltpu.VMEM((RPT,), jnp.float32),
        pltpu.VMEM((NBW, EB), jnp.int32),
        pltpu.VMEM((NBW, EB), jnp.float32),
        pltpu.SemaphoreType.DMA,
    ],
)
def _deg_kernel(dst2w_hbm, ew2w_hbm, out_hbm, acc, buf, didx, ewb, sem):
    cid = lax.axis_index("c")
    sid = lax.axis_index("s")
    for j in range(RPT // 16):
        buf[pl.ds(j * 16, 16)] = jnp.zeros((16,), jnp.float32)
    pltpu.sync_copy(buf, acc.at[pl.ds(sid * RPT, RPT)])

    pltpu.sync_copy(dst2w_hbm.at[pl.ds(sid * NBW, NBW), :], didx)
    pltpu.sync_copy(ew2w_hbm.at[pl.ds(sid * NBW, NBW), :], ewb)
    plsc.subcore_barrier()

    K = 5

    @pl.loop(0, NBW, step=K)
    def _edge_group(i):
        for k in range(K):
            pltpu.async_copy(ewb.at[i + k], acc.at[didx.at[i + k]], sem,
                             add=True)
        for k in range(K):
            pltpu.make_async_copy(ewb.at[i + k], acc.at[didx.at[i + k]],
                                  sem).wait()

    plsc.subcore_barrier()

    @pl.when(cid == 0)
    def _wb():
        pltpu.sync_copy(acc.at[pl.ds(sid * RPT, RPT)], buf)
        pltpu.sync_copy(buf, out_hbm.at[pl.ds(sid * RPT, RPT)])


def _make_agg(F, plan, NSLOT):
    CHMAX = max(ch for _, ch, _, _ in plan)
    assert sum(n * ch for n, ch, _, _ in plan) == NB

    @functools.partial(
        pl.kernel,
        out_type=jax.ShapeDtypeStruct((NC, NACC, F), jnp.float32),
        mesh=_mesh,
        compiler_params=pltpu.CompilerParams(use_tc_tiling_on_sc=False),
        scratch_types=(
            [
                pltpu.VMEM_SHARED((NACC, F), jnp.float32),
                pltpu.VMEM((16, F), jnp.float32),
                pltpu.VMEM((CHMAX, EB), jnp.int32),
                pltpu.VMEM((CHMAX, EB), jnp.int32),
                pltpu.VMEM((CHMAX, EB), jnp.float32),
            ]
            + [pltpu.VMEM((EB, F), jnp.float32)] * NSLOT
            + [pltpu.SemaphoreType.DMA] * (2 * NSLOT)
        ),
    )
    def _agg(hs0_hbm, hs1_hbm, src2_hbm, dst2_hbm, ew2_hbm, out_hbm,
             acc, zbuf, sidx, didx, ewb, *rest):
        rows = rest[:NSLOT]
        gsem = rest[NSLOT:2 * NSLOT]
        ssem = rest[2 * NSLOT:3 * NSLOT]
        cid = lax.axis_index("c")
        sid = lax.axis_index("s")

        for i in range(16):
            for j in range(F // 16):
                zbuf[i, pl.ds(j * 16, 16)] = jnp.zeros((16,), jnp.float32)

        @pl.loop(0, RPT // 16)
        def _zero(k):
            pltpu.sync_copy(zbuf, acc.at[pl.ds(sid * RPT + k * 16, 16), :])

        plsc.subcore_barrier()

        def _issue_gather(j, slot):
            @pl.when(cid == 0)
            def _g0():
                pltpu.async_copy(hs0_hbm.at[sidx.at[j]], rows[slot],
                                 gsem[slot])

            @pl.when(cid == 1)
            def _g1():
                pltpu.async_copy(hs1_hbm.at[sidx.at[j]], rows[slot],
                                 gsem[slot])

        def _wait_gather(j, slot):
            pltpu.make_async_copy(hs0_hbm.at[sidx.at[j]], rows[slot],
                                  gsem[slot]).wait()

        def _run_chunk(mrow, CH, nslot, L):
            pltpu.sync_copy(src2_hbm.at[pl.ds(mrow, CH), :], sidx.at[pl.ds(0, CH)])
            pltpu.sync_copy(dst2_hbm.at[pl.ds(mrow, CH), :], didx.at[pl.ds(0, CH)])
            pltpu.sync_copy(ew2_hbm.at[pl.ds(mrow, CH), :], ewb.at[pl.ds(0, CH)])

            for k in range(L):
                _issue_gather(k, k)

            @pl.loop(0, CH, step=nslot)
            def _edge_batch(i):
                for b in range(nslot):
                    j = i + b
                    _wait_gather(j, b)
                    nslot_b = (b + L) % nslot

                    @pl.when(j >= nslot - L)
                    def _ws():
                        pltpu.make_async_copy(rows[nslot_b],
                                              acc.at[didx.at[j]],
                                              ssem[nslot_b]).wait()

                    jn = jnp.minimum(j + L, CH - 1)
                    _issue_gather(jn, nslot_b)

                    for rb in range(EB // 16):
                        wv = ewb[j, pl.ds(rb * 16, 16)]
                        for rr in range(16):
                            r = rb * 16 + rr
                            w = wv[rr]
                            for c in range(F // 16):
                                rows[b][r, pl.ds(c * 16, 16)] = (
                                    rows[b][r, pl.ds(c * 16, 16)] * w)

                    pltpu.async_copy(rows[b], acc.at[didx.at[j]], ssem[b],
                                     add=True)

            for k in range(L):
                pltpu.make_async_copy(hs0_hbm.at[sidx.at[CH - 1]], rows[k],
                                      gsem[k]).wait()
            for k in range(nslot - L):
                slot = (L + k) % nslot
                pltpu.make_async_copy(rows[slot], acc.at[didx.at[CH - 1]],
                                      ssem[slot]).wait()

        base = 0
        for (nch, CH, nslot, L) in plan:
            if nch == 1:
                _run_chunk(sid * NB + base, CH, nslot, L)
            else:
                @pl.loop(0, nch)
                def _chunk(ci, base=base, CH=CH, nslot=nslot, L=L):
                    _run_chunk(sid * NB + base + ci * CH, CH, nslot, L)
            base += nch * CH

        plsc.subcore_barrier()

        @pl.loop(0, RPT // EB, step=2)
        def _writeback(i):
            for b in range(2):
                k = i + b
                r0 = sid * RPT + k * EB

                @pl.when(k >= 2)
                def _ww():
                    pltpu.make_async_copy(
                        rows[b], out_hbm.at[cid, pl.ds(r0, EB), :],
                        ssem[b]).wait()

                pltpu.sync_copy(acc.at[pl.ds(r0, EB), :], rows[b])
                pltpu.async_copy(rows[b], out_hbm.at[cid, pl.ds(r0, EB), :],
                                 ssem[b])

        for b in range(2):
            pltpu.make_async_copy(rows[b],
                                  out_hbm.at[cid, pl.ds(sid * RPT, EB), :],
                                  ssem[b]).wait()

    return _agg


_agg_l1 = _make_agg(DH // 2, [(5, 50, 2, 1)], 2)
_agg_l2 = _make_agg(DOUT // 2, [(1, 250, 5, 3)], 5)


_R = 2000
_G = N // _R


def _tc1_body(d, x, w1, hsa, hsb):
    dis = lax.rsqrt(d[...] + 1.0)
    h = jnp.dot(x[...], w1[...], preferred_element_type=jnp.float32)
    hs = h * dis
    hsa[...] = hs[:, : DH // 2]
    hsb[...] = hs[:, DH // 2:]


def _tc1(d, x, w1):
    return pl.pallas_call(
        _tc1_body,
        grid=(_G,),
        in_specs=[
            pl.BlockSpec((_R, 1), lambda i: (i, 0)),
            pl.BlockSpec((_R, DIN), lambda i: (i, 0)),
            pl.BlockSpec((DIN, DH), lambda i: (0, 0)),
        ],
        out_specs=[
            pl.BlockSpec((_R, DH // 2), lambda i: (i, 0)),
            pl.BlockSpec((_R, DH // 2), lambda i: (i, 0)),
        ],
        out_shape=[
            jax.ShapeDtypeStruct((N, DH // 2), jnp.float32),
            jax.ShapeDtypeStruct((N, DH // 2), jnp.float32),
        ],
    )(d, x, w1)


def _tc2_body(d, a0, a1, hsa, hsb, b1, w2, o0, o1):
    dis = lax.rsqrt(d[...] + 1.0)
    agg = jnp.concatenate([a0[...] + hsa[...], a1[...] + hsb[...]], axis=1)
    t = jnp.maximum(dis * agg + b1[...], 0.0)
    hs2 = jnp.dot(t, w2[...], preferred_element_type=jnp.float32) * dis
    o0[...] = hs2[:, : DOUT // 2]
    o1[...] = hs2[:, DOUT // 2:]


def _tc2(d, a0, a1, hsa, hsb, b1, w2):
    return pl.pallas_call(
        _tc2_body,
        grid=(_G,),
        in_specs=[
            pl.BlockSpec((_R, 1), lambda i: (i, 0)),
            pl.BlockSpec((_R, DH // 2), lambda i: (i, 0)),
            pl.BlockSpec((_R, DH // 2), lambda i: (i, 0)),
            pl.BlockSpec((_R, DH // 2), lambda i: (i, 0)),
            pl.BlockSpec((_R, DH // 2), lambda i: (i, 0)),
            pl.BlockSpec((1, DH), lambda i: (0, 0)),
            pl.BlockSpec((DH, DOUT), lambda i: (0, 0)),
        ],
        out_specs=[
            pl.BlockSpec((_R, DOUT // 2), lambda i: (i, 0)),
            pl.BlockSpec((_R, DOUT // 2), lambda i: (i, 0)),
        ],
        out_shape=[
            jax.ShapeDtypeStruct((N, DOUT // 2), jnp.float32),
            jax.ShapeDtypeStruct((N, DOUT // 2), jnp.float32),
        ],
    )(d, a0, a1, hsa, hsb, b1, w2)


def _tc3_body(d, a0, a1, hsa, hsb, b2, out):
    dis = lax.rsqrt(d[...] + 1.0)
    o = jnp.concatenate([a0[...] + hsa[...], a1[...] + hsb[...]], axis=1)
    o = dis * o + b2[...]
    m = jnp.max(o, axis=1, keepdims=True)
    z = o - m
    out[...] = z - jnp.log(jnp.sum(jnp.exp(z), axis=1, keepdims=True))


def _tc3(d, a0, a1, hsa, hsb, b2):
    return pl.pallas_call(
        _tc3_body,
        grid=(_G,),
        in_specs=[
            pl.BlockSpec((_R, 1), lambda i: (i, 0)),
            pl.BlockSpec((_R, DOUT // 2), lambda i: (i, 0)),
            pl.BlockSpec((_R, DOUT // 2), lambda i: (i, 0)),
            pl.BlockSpec((_R, DOUT // 2), lambda i: (i, 0)),
            pl.BlockSpec((_R, DOUT // 2), lambda i: (i, 0)),
            pl.BlockSpec((1, DOUT), lambda i: (0, 0)),
        ],
        out_specs=pl.BlockSpec((_R, DOUT), lambda i: (i, 0)),
        out_shape=jax.ShapeDtypeStruct((N, DOUT), jnp.float32),
    )(d, a0, a1, hsa, hsb, b2)


def kernel(x, edge_index, edge_weight, W1, b1, W2, b2):
    src = edge_index[0]
    dst = edge_index[1]
    src2 = src.reshape(E // EB, EB)
    dst2 = dst.reshape(E // EB, EB)
    ew2 = edge_weight.reshape(E // EB, EB)

    deg = _deg_kernel(dst2, ew2)
    d = deg[:N].reshape(N, 1)

    hsa, hsb = _tc1(d, x, W1)

    agg1 = _agg_l1(hsa, hsb, src2, dst2, ew2)
    hs2a, hs2b = _tc2(d, agg1[0, :N], agg1[1, :N],
                      hsa, hsb, b1.reshape(1, DH), W2)

    agg2 = _agg_l2(hs2a, hs2b, src2, dst2, ew2)
    return _tc3(d, agg2[0, :N], agg2[1, :N],
                hs2a, hs2b, b2.reshape(1, DOUT))
